# baseline (speedup 1.0000x reference)
import functools

import jax
import jax.numpy as jnp
from jax import lax
from jax.experimental import pallas as pl
from jax.experimental.pallas import tpu as pltpu
from jax.experimental.pallas import tpu_sc as plsc

NUM_POSITIONS = 8
CH = 16
ROW = NUM_POSITIONS * CH
CHUNK = 512


def kernel(x, scales):
    B = x.shape[0]
    xf = x.reshape(B * ROW)
    NC, NS = 2, 16
    NW = NC * NS
    rows_per_w = B // NW
    n_chunks = rows_per_w // CHUNK
    assert rows_per_w * NW == B and n_chunks * CHUNK == rows_per_w
    scb = jnp.broadcast_to(scales[:, None], (NUM_POSITIONS, CH)).reshape(-1)
    scb = scb.astype(jnp.float32)

    mesh = plsc.VectorSubcoreMesh(
        core_axis_name="c", subcore_axis_name="s", num_cores=NC, num_subcores=NS
    )

    @functools.partial(
        pl.kernel,
        out_type=jax.ShapeDtypeStruct((B * ROW,), jnp.float32),
        mesh=mesh,
        scratch_types=[
            pltpu.VMEM((CHUNK * ROW,), jnp.float32),
            pltpu.VMEM((NUM_POSITIONS * CH,), jnp.float32),
        ],
        compiler_params=pltpu.CompilerParams(needs_layout_passes=False),
    )
    def sc_k(x_hbm, scb_hbm, out_hbm, buf, scv):
        wid = lax.axis_index("s") * NC + lax.axis_index("c")
        base = wid * (rows_per_w * ROW)
        pltpu.sync_copy(scb_hbm, scv)
        row_iota = lax.iota(jnp.int32, 16) * ROW

        def chunk_body(c, carry):
            start = base + c * (CHUNK * ROW)
            pltpu.sync_copy(x_hbm.at[pl.ds(start, CHUNK * ROW)], buf)

            def group_body(g, carry2):
                rows = row_iota + g * (16 * ROW)
                a = jnp.zeros((16,), jnp.float32)
                b = jnp.zeros((16,), jnp.float32)
                for p in range(NUM_POSITIONS):
                    sp = scv[pl.ds(CH * p, CH)]
                    a = a + sp * plsc.load_gather(buf, [rows + CH * p])
                    b = b + sp * plsc.load_gather(buf, [rows + (CH * p + 1)])
                for p in range(NUM_POSITIONS):
                    plsc.store_scatter(buf, [rows + (CH * p + 12)], a)
                    plsc.store_scatter(buf, [rows + (CH * p + 13)], b)
                return carry2

            lax.fori_loop(0, CHUNK // 16, group_body, 0)
            pltpu.sync_copy(buf, out_hbm.at[pl.ds(start, CHUNK * ROW)])
            return carry

        lax.fori_loop(0, n_chunks, chunk_body, 0)

    yf = sc_k(xf, scb)
    return yf.reshape(B, NUM_POSITIONS, CH)


# speedup vs baseline: 3.8393x; 3.8393x over previous
import functools

import jax
import jax.numpy as jnp
from jax import lax
from jax.experimental import pallas as pl
from jax.experimental.pallas import tpu as pltpu
from jax.experimental.pallas import tpu_sc as plsc

NUM_POSITIONS = 8
CH = 16
ROW = NUM_POSITIONS * CH
CHUNK = 512


def kernel(x, scales):
    B = x.shape[0]
    x2 = x.reshape(B, ROW)
    NC, NS = 2, 16
    NW = NC * NS
    rows_per_w = B // NW
    n_chunks = rows_per_w // CHUNK
    assert rows_per_w * NW == B and n_chunks * CHUNK == rows_per_w
    scb = jnp.broadcast_to(scales[:, None], (NUM_POSITIONS, CH)).reshape(1, ROW)
    scb = scb.astype(jnp.float32)

    mesh = plsc.VectorSubcoreMesh(
        core_axis_name="c", subcore_axis_name="s", num_cores=NC, num_subcores=NS
    )

    @functools.partial(
        pl.kernel,
        out_type=jax.ShapeDtypeStruct((B, ROW), jnp.float32),
        mesh=mesh,
        scratch_types=[
            pltpu.VMEM((CHUNK, ROW), jnp.float32),
            pltpu.VMEM((1, ROW), jnp.float32),
        ],
        compiler_params=pltpu.CompilerParams(needs_layout_passes=False),
    )
    def sc_k(x_hbm, scb_hbm, out_hbm, buf, scv):
        wid = lax.axis_index("s") * NC + lax.axis_index("c")
        base = wid * rows_per_w
        pltpu.sync_copy(scb_hbm, scv)
        row_iota = lax.iota(jnp.int32, 16)

        def chunk_body(c, carry):
            start = base + c * CHUNK
            pltpu.sync_copy(x_hbm.at[pl.ds(start, CHUNK)], buf)

            def group_body(g, carry2):
                rows = row_iota + g * 16
                a = jnp.zeros((16,), jnp.float32)
                b = jnp.zeros((16,), jnp.float32)
                for p in range(NUM_POSITIONS):
                    sp = scv[0, pl.ds(CH * p, CH)]
                    ca = jnp.full((16,), CH * p, jnp.int32)
                    cb = jnp.full((16,), CH * p + 1, jnp.int32)
                    a = a + sp * plsc.load_gather(buf, [rows, ca])
                    b = b + sp * plsc.load_gather(buf, [rows, cb])
                for p in range(NUM_POSITIONS):
                    c12 = jnp.full((16,), CH * p + 12, jnp.int32)
                    c13 = jnp.full((16,), CH * p + 13, jnp.int32)
                    plsc.store_scatter(buf, [rows, c12], a)
                    plsc.store_scatter(buf, [rows, c13], b)
                return carry2

            lax.fori_loop(0, CHUNK // 16, group_body, 0)
            pltpu.sync_copy(buf, out_hbm.at[pl.ds(start, CHUNK)])
            return carry

        lax.fori_loop(0, n_chunks, chunk_body, 0)

    y2 = sc_k(x2, scb)
    return y2.reshape(B, NUM_POSITIONS, CH)


# speedup vs baseline: 5.4784x; 1.4269x over previous
import jax
import jax.numpy as jnp
from jax import lax
from jax.experimental import pallas as pl
from jax.experimental.pallas import tpu as pltpu

NUM_POSITIONS = 8
CH = 16
ROW = NUM_POSITIONS * CH

BLK = 4096
NBUF = 4


def _compute(blk, wa, wb):
    a = jnp.sum(blk * wa, axis=1, keepdims=True)
    b = jnp.sum(blk * wb, axis=1, keepdims=True)
    lane = lax.broadcasted_iota(jnp.int32, (1, ROW), 1) % CH
    out = jnp.where(lane == 12, a, blk)
    return jnp.where(lane == 13, b, out)


def _body(wa_ref, wb_ref, x_hbm, o_hbm, ibuf, obuf, isem, osem):
    nblocks = x_hbm.shape[0] // BLK
    wa = wa_ref[...]
    wb = wb_ref[...]

    def in_copy(i, slot):
        return pltpu.make_async_copy(
            x_hbm.at[pl.ds(i * BLK, BLK), :], ibuf.at[slot], isem.at[slot]
        )

    def out_copy(i, slot):
        return pltpu.make_async_copy(
            obuf.at[slot], o_hbm.at[pl.ds(i * BLK, BLK), :], osem.at[slot]
        )

    for j in range(NBUF):
        in_copy(j, j).start()

    def step(i, carry):
        slot = lax.rem(i, NBUF)
        in_copy(i, slot).wait()

        @pl.when(i >= NBUF)
        def _():
            out_copy(i - NBUF, slot).wait()

        obuf[slot] = _compute(ibuf[slot], wa, wb)
        out_copy(i, slot).start()

        @pl.when(i + NBUF < nblocks)
        def _():
            in_copy(i + NBUF, slot).start()

        return carry

    lax.fori_loop(0, nblocks, step, 0)
    for j in range(NBUF):
        i = nblocks - NBUF + j
        out_copy(i, lax.rem(jnp.int32(i), NBUF)).wait()


def kernel(x, scales):
    B = x.shape[0]
    x2 = x.reshape(B, ROW)
    lane = lax.iota(jnp.int32, ROW)
    pos = lane // CH
    ch = lane % CH
    sc = scales[pos]
    wa = jnp.where(ch == 0, sc, 0.0).reshape(1, ROW)
    wb = jnp.where(ch == 1, sc, 0.0).reshape(1, ROW)
    y2 = pl.pallas_call(
        _body,
        in_specs=[
            pl.BlockSpec((1, ROW), lambda: (0, 0)),
            pl.BlockSpec((1, ROW), lambda: (0, 0)),
            pl.BlockSpec(memory_space=pltpu.HBM),
        ],
        out_specs=pl.BlockSpec(memory_space=pltpu.HBM),
        out_shape=jax.ShapeDtypeStruct((B, ROW), x.dtype),
        scratch_shapes=[
            pltpu.VMEM((NBUF, BLK, ROW), jnp.float32),
            pltpu.VMEM((NBUF, BLK, ROW), jnp.float32),
            pltpu.SemaphoreType.DMA((NBUF,)),
            pltpu.SemaphoreType.DMA((NBUF,)),
        ],
    )(wa, wb, x2)
    return y2.reshape(B, NUM_POSITIONS, CH)


# speedup vs baseline: 13.4128x; 2.4483x over previous
import jax
import jax.numpy as jnp
from jax import lax
from jax.experimental import pallas as pl
from jax.experimental.pallas import tpu as pltpu

NUM_POSITIONS = 8
CH = 16
ROW = NUM_POSITIONS * CH

BB = 2048


def _body(s_ref, x_ref, o_ref):
    blk = x_ref[...]
    a = jnp.zeros((1, BB), jnp.float32)
    b = jnp.zeros((1, BB), jnp.float32)
    for q in range(NUM_POSITIONS):
        a = a + s_ref[q] * blk[CH * q : CH * q + 1, :]
        b = b + s_ref[q] * blk[CH * q + 1 : CH * q + 2, :]
    rid = lax.broadcasted_iota(jnp.int32, (ROW, 1), 0) % CH
    out = jnp.where(rid == 12, a, blk)
    out = jnp.where(rid == 13, b, out)
    o_ref[...] = out


def kernel(x, scales):
    B = x.shape[0]
    xt = jnp.transpose(x, (1, 2, 0)).reshape(ROW, B)
    grid = (B // BB,)
    yt = pl.pallas_call(
        _body,
        grid=grid,
        in_specs=[
            pl.BlockSpec(memory_space=pltpu.SMEM),
            pl.BlockSpec((ROW, BB), lambda i: (0, i)),
        ],
        out_specs=pl.BlockSpec((ROW, BB), lambda i: (0, i)),
        out_shape=jax.ShapeDtypeStruct((ROW, B), x.dtype),
    )(scales, xt)
    return jnp.transpose(yt.reshape(NUM_POSITIONS, CH, B), (2, 0, 1))


# speedup vs baseline: 18.9710x; 1.4144x over previous
import jax
import jax.numpy as jnp
from jax import lax
from jax.experimental import pallas as pl
from jax.experimental.pallas import tpu as pltpu

NUM_POSITIONS = 8
CH = 16
ROW = NUM_POSITIONS * CH

BB = 4096


def _body(s_ref, x_ref, o_ref):
    blk = x_ref[...]
    a = jnp.zeros((1, BB), jnp.float32)
    b = jnp.zeros((1, BB), jnp.float32)
    for q in range(NUM_POSITIONS):
        a = a + s_ref[q] * blk[CH * q : CH * q + 1, :]
        b = b + s_ref[q] * blk[CH * q + 1 : CH * q + 2, :]
    rid = lax.broadcasted_iota(jnp.int32, (ROW, 1), 0) % CH
    out = jnp.where(rid == 12, a, blk)
    out = jnp.where(rid == 13, b, out)
    o_ref[...] = out


def kernel(x, scales):
    B = x.shape[0]
    xt = jnp.transpose(x, (1, 2, 0)).reshape(ROW, B)
    grid = (B // BB,)
    yt = pl.pallas_call(
        _body,
        grid=grid,
        in_specs=[
            pl.BlockSpec(memory_space=pltpu.SMEM),
            pl.BlockSpec((ROW, BB), lambda i: (0, i)),
        ],
        out_specs=pl.BlockSpec((ROW, BB), lambda i: (0, i)),
        out_shape=jax.ShapeDtypeStruct((ROW, B), x.dtype),
    )(scales, xt)
    return jnp.transpose(yt.reshape(NUM_POSITIONS, CH, B), (2, 0, 1))


# speedup vs baseline: 20.6296x; 1.0874x over previous
import jax
import jax.numpy as jnp
from jax import lax
from jax.experimental import pallas as pl
from jax.experimental.pallas import tpu as pltpu

NUM_POSITIONS = 8
CH = 16
ROW = NUM_POSITIONS * CH

BB = 8192


def _body(s_ref, x_ref, o_ref):
    blk = x_ref[...]
    a = jnp.zeros((1, BB), jnp.float32)
    b = jnp.zeros((1, BB), jnp.float32)
    for q in range(NUM_POSITIONS):
        a = a + s_ref[q] * blk[CH * q : CH * q + 1, :]
        b = b + s_ref[q] * blk[CH * q + 1 : CH * q + 2, :]
    rid = lax.broadcasted_iota(jnp.int32, (ROW, 1), 0) % CH
    out = jnp.where(rid == 12, a, blk)
    out = jnp.where(rid == 13, b, out)
    o_ref[...] = out


def kernel(x, scales):
    B = x.shape[0]
    xt = jnp.transpose(x, (1, 2, 0)).reshape(ROW, B)
    grid = (B // BB,)
    yt = pl.pallas_call(
        _body,
        grid=grid,
        in_specs=[
            pl.BlockSpec(memory_space=pltpu.SMEM),
            pl.BlockSpec((ROW, BB), lambda i: (0, i)),
        ],
        out_specs=pl.BlockSpec((ROW, BB), lambda i: (0, i)),
        out_shape=jax.ShapeDtypeStruct((ROW, B), x.dtype),
    )(scales, xt)
    return jnp.transpose(yt.reshape(NUM_POSITIONS, CH, B), (2, 0, 1))


# speedup vs baseline: 21.2091x; 1.0281x over previous
import jax
import jax.numpy as jnp
from jax import lax
from jax.experimental import pallas as pl
from jax.experimental.pallas import tpu as pltpu

NUM_POSITIONS = 8
CH = 16
ROW = NUM_POSITIONS * CH

BB = 16384


def _body(s_ref, x_ref, o_ref):
    blk = x_ref[...]
    a = jnp.zeros((1, BB), jnp.float32)
    b = jnp.zeros((1, BB), jnp.float32)
    for q in range(NUM_POSITIONS):
        a = a + s_ref[q] * blk[CH * q : CH * q + 1, :]
        b = b + s_ref[q] * blk[CH * q + 1 : CH * q + 2, :]
    rid = lax.broadcasted_iota(jnp.int32, (ROW, 1), 0) % CH
    out = jnp.where(rid == 12, a, blk)
    out = jnp.where(rid == 13, b, out)
    o_ref[...] = out


def kernel(x, scales):
    B = x.shape[0]
    xt = jnp.transpose(x, (1, 2, 0)).reshape(ROW, B)
    grid = (B // BB,)
    yt = pl.pallas_call(
        _body,
        grid=grid,
        in_specs=[
            pl.BlockSpec(memory_space=pltpu.SMEM),
            pl.BlockSpec((ROW, BB), lambda i: (0, i)),
        ],
        out_specs=pl.BlockSpec((ROW, BB), lambda i: (0, i)),
        out_shape=jax.ShapeDtypeStruct((ROW, B), x.dtype),
    )(scales, xt)
    return jnp.transpose(yt.reshape(NUM_POSITIONS, CH, B), (2, 0, 1))
